# baseline (speedup 1.0000x reference)
import functools

import jax
import jax.numpy as jnp
import numpy as np
from jax import lax
from jax.experimental import pallas as pl
from jax.experimental.pallas import tpu as pltpu

N_GRAPHS = 128
NPG = 64
EPG = 256
D_NODE = 128
D_EDGE = 16
H = 32
HEADS = 4
DK = H // HEADS
STEPS = 3


def _ln(v, g, b, eps=1e-6):
    mu = jnp.mean(v, axis=-1, keepdims=True)
    var = jnp.mean((v - mu) ** 2, axis=-1, keepdims=True)
    return (v - mu) / jnp.sqrt(var + eps) * g + b


def _gelu(x):
    return 0.5 * x * (1.0 + lax.erf(x * np.float32(1.0 / np.sqrt(2.0))))


def _body(x_ref, ea_ref, src_ref, dst_ref,
          Wp_ref, bp_ref, We1_ref, be1_ref, We2_ref, be2_ref, bc_ref,
          Wi_ref, bi_ref, Wh_ref, bh_ref, Wq_ref, Wk_ref, Wv_ref,
          g1_ref, c1_ref, W1_ref, b1_ref, W2_ref, b2_ref, g2_ref, c2_ref,
          out_ref):
    f32 = jnp.float32
    dot_d = functools.partial(lax.dot_general, preferred_element_type=f32)
    dot_x = functools.partial(lax.dot_general, preferred_element_type=f32,
                              precision=lax.Precision.HIGHEST)

    def mm(a, b):
        return dot_d(a, b, (((1,), (0,)), ((), ())))

    h = jnp.maximum(mm(x_ref[...], Wp_ref[...]) + bp_ref[...], 0.0)
    z = jnp.maximum(mm(ea_ref[...], We1_ref[...]) + be1_ref[...], 0.0)
    ew = mm(z, We2_ref[...]) + be2_ref[...]
    ewb = ew.astype(jnp.bfloat16).astype(f32)
    src = src_ref[0, 0, :]
    dst = dst_ref[0, 0, :]
    SgT = (lax.broadcasted_iota(jnp.int32, (NPG, EPG), 0) == src[None, :]
           ).astype(f32)
    DgT = (lax.broadcasted_iota(jnp.int32, (NPG, EPG), 0) == dst[None, :]
           ).astype(f32)

    Wi = Wi_ref[...]
    Wh = Wh_ref[...]

    hidden = h
    for _ in range(STEPS):
        hb = h.astype(jnp.bfloat16).astype(f32)
        h_s = dot_x(SgT, hb, (((0,), (0,)), ((), ())))
        m = h_s[:, 0:1] * ewb[:, 0:H]
        for i in range(1, H):
            m = m + h_s[:, i:i + 1] * ewb[:, i * H:(i + 1) * H]
        agg = dot_x(DgT, m, (((1,), (0,)), ((), ())))
        a = jnp.maximum(agg + bc_ref[...], 0.0)
        gi = mm(a, Wi) + bi_ref[...]
        gh = mm(hidden, Wh) + bh_ref[...]
        r = jax.nn.sigmoid(gi[:, :H] + gh[:, :H])
        zg = jax.nn.sigmoid(gi[:, H:2 * H] + gh[:, H:2 * H])
        nn_ = jnp.tanh(gi[:, 2 * H:] + r * gh[:, 2 * H:])
        hidden = (1.0 - zg) * nn_ + zg * hidden
        h = hidden

    q = mm(h, Wq_ref[...])
    k = mm(h, Wk_ref[...])
    v = mm(h, Wv_ref[...])
    scale = np.float32(1.0 / np.sqrt(DK))
    outs = []
    for hd in range(HEADS):
        sl = slice(hd * DK, (hd + 1) * DK)
        s = dot_d(q[:, sl], k[:, sl], (((1,), (1,)), ((), ()))) * scale
        s = s - jnp.max(s, axis=-1, keepdims=True)
        e = jnp.exp(s)
        att = e / jnp.sum(e, axis=-1, keepdims=True)
        outs.append(mm(att, v[:, sl]))
    o = jnp.concatenate(outs, axis=1)

    o = _ln(o + h, g1_ref[...], c1_ref[...])
    ff = mm(_gelu(mm(o, W1_ref[...]) + b1_ref[...]), W2_ref[...]) + b2_ref[...]
    o2 = _ln(o + ff, g2_ref[...], c2_ref[...])
    out_ref[0, :, :] = o2


def kernel(x, edge_attr, edge_index, W_proj, b_proj, We1, be1, We2, be2,
           b_conv, Wi, bi, Wh, bh, Wq, Wk, Wv, ln1_g, ln1_b, W1, b1, W2, b2,
           ln2_g, ln2_b):
    src = (edge_index[0] % NPG).astype(jnp.int32).reshape(N_GRAPHS, 1, EPG)
    dst = (edge_index[1] % NPG).astype(jnp.int32).reshape(N_GRAPHS, 1, EPG)
    row = lambda t: t.reshape(1, -1)

    grid = (N_GRAPHS,)
    full = lambda shp: pl.BlockSpec(shp, lambda g: (0,) * len(shp))
    in_specs = [
        pl.BlockSpec((NPG, D_NODE), lambda g: (g, 0)),
        pl.BlockSpec((EPG, D_EDGE), lambda g: (g, 0)),
        pl.BlockSpec((1, 1, EPG), lambda g: (g, 0, 0)),
        pl.BlockSpec((1, 1, EPG), lambda g: (g, 0, 0)),
        full((D_NODE, H)), full((1, H)),
        full((D_EDGE, H)), full((1, H)),
        full((H, H * H)), full((1, H * H)),
        full((1, H)),
        full((H, 3 * H)), full((1, 3 * H)),
        full((H, 3 * H)), full((1, 3 * H)),
        full((H, H)), full((H, H)), full((H, H)),
        full((1, H)), full((1, H)),
        full((H, 2 * H)), full((1, 2 * H)),
        full((2 * H, H)), full((1, H)),
        full((1, H)), full((1, H)),
    ]
    out = pl.pallas_call(
        _body,
        grid=grid,
        in_specs=in_specs,
        out_specs=pl.BlockSpec((1, NPG, H), lambda g: (g, 0, 0)),
        out_shape=jax.ShapeDtypeStruct((N_GRAPHS, NPG, H), jnp.float32),
        compiler_params=pltpu.CompilerParams(
            dimension_semantics=("arbitrary",)),
    )(x, edge_attr, src, dst,
      W_proj, row(b_proj), We1, row(be1), We2, row(be2), row(b_conv),
      Wi, row(bi), Wh, row(bh), Wq, Wk, Wv,
      row(ln1_g), row(ln1_b), W1, row(b1), W2, row(b2),
      row(ln2_g), row(ln2_b))
    return out


# speedup vs baseline: 4.7727x; 4.7727x over previous
import functools

import jax
import jax.numpy as jnp
import numpy as np
from jax import lax
from jax.experimental import pallas as pl
from jax.experimental.pallas import tpu as pltpu

N_GRAPHS = 128
NPG = 64
EPG = 256
D_NODE = 128
D_EDGE = 16
H = 32
HEADS = 4
DK = H // HEADS
STEPS = 3
G = 8
NB = G * NPG
EB = G * EPG


def _ln(v, g, b, eps=1e-6):
    mu = jnp.mean(v, axis=-1, keepdims=True)
    var = jnp.mean((v - mu) ** 2, axis=-1, keepdims=True)
    return (v - mu) / jnp.sqrt(var + eps) * g + b


def _gelu(x):
    return 0.5 * x * (1.0 + lax.erf(x * np.float32(1.0 / np.sqrt(2.0))))


def _body(x_ref, ea_ref, src_ref, dst_ref,
          Wp_ref, bp_ref, We1_ref, be1_ref, We2_ref, be2_ref, bc_ref,
          Wi_ref, bi_ref, Wh_ref, bh_ref, Wq_ref, Wk_ref, Wv_ref,
          g1_ref, c1_ref, W1_ref, b1_ref, W2_ref, b2_ref, g2_ref, c2_ref,
          out_ref):
    f32 = jnp.float32
    bf16 = jnp.bfloat16
    dot_d = functools.partial(lax.dot_general, preferred_element_type=f32)
    dot_x = functools.partial(lax.dot_general, preferred_element_type=f32,
                              precision=lax.Precision.HIGHEST)

    def mm(a, b):
        return dot_d(a, b, (((1,), (0,)), ((), ())))

    h = jnp.maximum(mm(x_ref[...], Wp_ref[...]) + bp_ref[...], 0.0)
    z = jnp.maximum(mm(ea_ref[...], We1_ref[...]) + be1_ref[...], 0.0)
    ewb = (mm(z, We2_ref[...]) + be2_ref[...]).astype(bf16).astype(f32)

    src = src_ref[0, 0, :]
    dst = dst_ref[0, 0, :]
    iota_ne = lax.broadcasted_iota(jnp.int32, (NPG, EPG), 0)
    SgT = []
    DgT = []
    for g in range(G):
        es = slice(g * EPG, (g + 1) * EPG)
        SgT.append((iota_ne == src[None, es]).astype(f32))
        DgT.append((iota_ne == dst[None, es]).astype(f32))
    T = (lax.broadcasted_iota(jnp.int32, (H, H * H), 1) // H
         == lax.broadcasted_iota(jnp.int32, (H, H * H), 0)).astype(f32)

    Wi = Wi_ref[...]
    Wh = Wh_ref[...]

    hidden = h
    for _ in range(STEPS):
        hb = hidden.astype(bf16).astype(f32)
        hs = jnp.concatenate(
            [dot_d(SgT[g], hb[g * NPG:(g + 1) * NPG], (((0,), (0,)), ((), ())))
             for g in range(G)], axis=0)
        hse = mm(hs, T)
        prod = hse * ewb
        r = prod
        w = H * H
        while w > H:
            w //= 2
            r = r[:, :w] + r[:, w:2 * w]
        m = r
        agg = jnp.concatenate(
            [dot_x(DgT[g], m[g * EPG:(g + 1) * EPG], (((1,), (0,)), ((), ())))
             for g in range(G)], axis=0)
        a = jnp.maximum(agg + bc_ref[...], 0.0)
        gi = mm(a, Wi) + bi_ref[...]
        gh = mm(hidden, Wh) + bh_ref[...]
        rg = jax.nn.sigmoid(gi[:, :H] + gh[:, :H])
        zg = jax.nn.sigmoid(gi[:, H:2 * H] + gh[:, H:2 * H])
        nn_ = jnp.tanh(gi[:, 2 * H:] + rg * gh[:, 2 * H:])
        hidden = (1.0 - zg) * nn_ + zg * hidden

    h = hidden
    q = mm(h, Wq_ref[...])
    k = mm(h, Wk_ref[...])
    v = mm(h, Wv_ref[...])
    scale = np.float32(1.0 / np.sqrt(DK))
    og = []
    for g in range(G):
        ns = slice(g * NPG, (g + 1) * NPG)
        outs = []
        for hd in range(HEADS):
            sl = slice(hd * DK, (hd + 1) * DK)
            s = dot_d(q[ns, sl], k[ns, sl], (((1,), (1,)), ((), ()))) * scale
            s = s - jnp.max(s, axis=-1, keepdims=True)
            e = jnp.exp(s)
            att = e / jnp.sum(e, axis=-1, keepdims=True)
            outs.append(mm(att, v[ns, sl]))
        og.append(jnp.concatenate(outs, axis=1))
    o = jnp.concatenate(og, axis=0)

    o = _ln(o + h, g1_ref[...], c1_ref[...])
    ff = mm(_gelu(mm(o, W1_ref[...]) + b1_ref[...]), W2_ref[...]) + b2_ref[...]
    o2 = _ln(o + ff, g2_ref[...], c2_ref[...])
    out_ref[...] = o2.reshape(G, NPG, H)


def kernel(x, edge_attr, edge_index, W_proj, b_proj, We1, be1, We2, be2,
           b_conv, Wi, bi, Wh, bh, Wq, Wk, Wv, ln1_g, ln1_b, W1, b1, W2, b2,
           ln2_g, ln2_b):
    n_prog = N_GRAPHS // G
    src = (edge_index[0] % NPG).astype(jnp.int32).reshape(n_prog, 1, EB)
    dst = (edge_index[1] % NPG).astype(jnp.int32).reshape(n_prog, 1, EB)
    row = lambda t: t.reshape(1, -1)

    full = lambda shp: pl.BlockSpec(shp, lambda g: (0,) * len(shp))
    in_specs = [
        pl.BlockSpec((NB, D_NODE), lambda g: (g, 0)),
        pl.BlockSpec((EB, D_EDGE), lambda g: (g, 0)),
        pl.BlockSpec((1, 1, EB), lambda g: (g, 0, 0)),
        pl.BlockSpec((1, 1, EB), lambda g: (g, 0, 0)),
        full((D_NODE, H)), full((1, H)),
        full((D_EDGE, H)), full((1, H)),
        full((H, H * H)), full((1, H * H)),
        full((1, H)),
        full((H, 3 * H)), full((1, 3 * H)),
        full((H, 3 * H)), full((1, 3 * H)),
        full((H, H)), full((H, H)), full((H, H)),
        full((1, H)), full((1, H)),
        full((H, 2 * H)), full((1, 2 * H)),
        full((2 * H, H)), full((1, H)),
        full((1, H)), full((1, H)),
    ]
    out = pl.pallas_call(
        _body,
        grid=(n_prog,),
        in_specs=in_specs,
        out_specs=pl.BlockSpec((G, NPG, H), lambda g: (g, 0, 0)),
        out_shape=jax.ShapeDtypeStruct((N_GRAPHS, NPG, H), jnp.float32),
        compiler_params=pltpu.CompilerParams(
            dimension_semantics=("arbitrary",)),
    )(x, edge_attr, src, dst,
      W_proj, row(b_proj), We1, row(be1), We2, row(be2), row(b_conv),
      Wi, row(bi), Wh, row(bh), Wq, Wk, Wv,
      row(ln1_g), row(ln1_b), W1, row(b1), W2, row(b2),
      row(ln2_g), row(ln2_b))
    return out
